# Initial kernel scaffold; baseline (speedup 1.0000x reference)
import functools

import jax
import jax.numpy as jnp
from jax import lax
from jax.experimental import pallas as pl
from jax.experimental.pallas import tpu as pltpu
from jax.experimental.pallas import tpu_sc as plsc

_VOCAB = 1000000
_EMBED = 64
_BATCH = 16384
_SEQ = 200
_ROWS = 128
_NUM_BLOCKS = _BATCH // _ROWS
_NC, _NS = 2, 16
_NW = _NC * _NS
_BPW = _NUM_BLOCKS // _NW


def _sc_body(xb_hbm, table_hbm, out_hbm, idx_v, acc_v, sem_idx, sem_g):
    wid = lax.axis_index("s") * _NC + lax.axis_index("c")

    pltpu.sync_copy(xb_hbm.at[wid * _BPW], idx_v.at[0])

    for t in range(_BPW):
        slot = t % 2
        if t + 1 < _BPW:
            idx_cp = pltpu.async_copy(
                xb_hbm.at[wid * _BPW + t + 1], idx_v.at[1 - slot], sem_idx)

        pltpu.async_copy(
            table_hbm.at[idx_v.at[slot, 0]], acc_v, sem_g).wait()

        def _fire(s, carry):
            pltpu.async_copy(
                table_hbm.at[idx_v.at[slot, s]], acc_v, sem_g, add=True)
            return carry
        lax.fori_loop(1, _SEQ, _fire, 0)

        def _drain(s, carry):
            pltpu.make_async_copy(
                table_hbm.at[idx_v.at[slot, 0]], acc_v, sem_g).wait()
            return carry
        lax.fori_loop(1, _SEQ, _drain, 0)

        pltpu.sync_copy(
            acc_v, out_hbm.at[pl.ds((wid * _BPW + t) * _ROWS, _ROWS)])
        if t + 1 < _BPW:
            idx_cp.wait()


def _sc_pool(xb, table):
    mesh = plsc.VectorSubcoreMesh(core_axis_name="c", subcore_axis_name="s")
    f = pl.kernel(
        _sc_body,
        out_type=jax.ShapeDtypeStruct((_BATCH, _EMBED), jnp.float32),
        mesh=mesh,
        scratch_types=[
            pltpu.VMEM((2, _SEQ, _ROWS), jnp.int32),
            pltpu.VMEM((_ROWS, _EMBED), jnp.float32),
            pltpu.SemaphoreType.DMA,
            pltpu.SemaphoreType.DMA,
        ],
    )
    return f(xb, table)


def _tc_head_body(x_ref, summed_ref, w1t_ref, b1_ref, w2t_ref, b2_ref, o_ref):
    cnt = jnp.sum((x_ref[...] != 0).astype(jnp.float32), axis=1, keepdims=True)
    pooled = summed_ref[...] / jnp.maximum(cnt, 1.0)
    h = jnp.dot(pooled, w1t_ref[...], preferred_element_type=jnp.float32)
    h = jnp.maximum(h + b1_ref[...], 0.0)
    o_ref[...] = (
        jnp.dot(h, w2t_ref[...], preferred_element_type=jnp.float32)
        + b2_ref[...])


def _tc_head(x, summed, w1t, b1, w2t, b2):
    blk = 2048
    grid = (_BATCH // blk,)
    return pl.pallas_call(
        _tc_head_body,
        grid=grid,
        in_specs=[
            pl.BlockSpec((blk, _SEQ), lambda i: (i, 0)),
            pl.BlockSpec((blk, _EMBED), lambda i: (i, 0)),
            pl.BlockSpec((_EMBED, _EMBED), lambda i: (0, 0)),
            pl.BlockSpec((1, _EMBED), lambda i: (0, 0)),
            pl.BlockSpec((_EMBED, 1), lambda i: (0, 0)),
            pl.BlockSpec((1, 1), lambda i: (0, 0)),
        ],
        out_specs=pl.BlockSpec((blk, 1), lambda i: (i, 0)),
        out_shape=jax.ShapeDtypeStruct((_BATCH, 1), jnp.float32),
    )(x, summed, w1t, b1, w2t, b2)


def kernel(x, table, W1, b1, W2, b2):
    xb = x.reshape(_NUM_BLOCKS, _ROWS, _SEQ).swapaxes(1, 2)
    summed = _sc_pool(xb, table)
    return _tc_head(x, summed, W1.T, b1.reshape(1, _EMBED),
                    W2.T, b2.reshape(1, 1))


# speedup vs baseline: 3.8408x; 3.8408x over previous
import functools

import jax
import jax.numpy as jnp
from jax import lax
from jax.experimental import pallas as pl
from jax.experimental.pallas import tpu as pltpu
from jax.experimental.pallas import tpu_sc as plsc

_VOCAB = 1000000
_EMBED = 64
_BATCH = 16384
_SEQ = 200
_ROWS = 128
_NUM_BLOCKS = _BATCH // _ROWS
_NC, _NS = 2, 16
_NW = _NC * _NS
_BPW = _NUM_BLOCKS // _NW


def _sc_body(xb_hbm, table_hbm, out_hbm, idx_v, acc_v, sem_idx, sem_g):
    wid = lax.axis_index("s") * _NC + lax.axis_index("c")

    pltpu.sync_copy(xb_hbm.at[wid * _BPW], idx_v.at[0])

    for t in range(_BPW):
        slot = t % 2
        if t + 1 < _BPW:
            idx_cp = pltpu.async_copy(
                xb_hbm.at[wid * _BPW + t + 1], idx_v.at[1 - slot], sem_idx)

        pltpu.async_copy(
            table_hbm.at[idx_v.at[slot, 0]], acc_v, sem_g).wait()

        def _fire(s, carry):
            pltpu.async_copy(
                table_hbm.at[idx_v.at[slot, s]], acc_v, sem_g, add=True)
            return carry
        lax.fori_loop(1, _SEQ, _fire, 0)

        def _drain(s, carry):
            pltpu.make_async_copy(
                table_hbm.at[idx_v.at[slot, 0]], acc_v, sem_g).wait()
            return carry
        lax.fori_loop(1, _SEQ, _drain, 0)

        pltpu.sync_copy(
            acc_v, out_hbm.at[pl.ds((wid * _BPW + t) * _ROWS, _ROWS)])
        if t + 1 < _BPW:
            idx_cp.wait()


def _sc_pool(xb, table):
    mesh = plsc.VectorSubcoreMesh(core_axis_name="c", subcore_axis_name="s")
    f = pl.kernel(
        _sc_body,
        out_type=jax.ShapeDtypeStruct((_BATCH, _EMBED), jnp.float32),
        mesh=mesh,
        scratch_types=[
            pltpu.VMEM((2, _SEQ, _ROWS), jnp.int32),
            pltpu.VMEM((_ROWS, _EMBED), jnp.float32),
            pltpu.SemaphoreType.DMA,
            pltpu.SemaphoreType.DMA,
        ],
        compiler_params=pltpu.CompilerParams(use_tc_tiling_on_sc=False),
    )
    return f(xb, table)


def _tc_head_body(x_ref, summed_ref, w1t_ref, b1_ref, w2t_ref, b2_ref, o_ref):
    cnt = jnp.sum((x_ref[...] != 0).astype(jnp.float32), axis=1, keepdims=True)
    pooled = summed_ref[...] / jnp.maximum(cnt, 1.0)
    h = jnp.dot(pooled, w1t_ref[...], preferred_element_type=jnp.float32)
    h = jnp.maximum(h + b1_ref[...], 0.0)
    o_ref[...] = (
        jnp.dot(h, w2t_ref[...], preferred_element_type=jnp.float32)
        + b2_ref[...])


def _tc_head(x, summed, w1t, b1, w2t, b2):
    blk = 2048
    grid = (_BATCH // blk,)
    return pl.pallas_call(
        _tc_head_body,
        grid=grid,
        in_specs=[
            pl.BlockSpec((blk, _SEQ), lambda i: (i, 0)),
            pl.BlockSpec((blk, _EMBED), lambda i: (i, 0)),
            pl.BlockSpec((_EMBED, _EMBED), lambda i: (0, 0)),
            pl.BlockSpec((1, _EMBED), lambda i: (0, 0)),
            pl.BlockSpec((_EMBED, 1), lambda i: (0, 0)),
            pl.BlockSpec((1, 1), lambda i: (0, 0)),
        ],
        out_specs=pl.BlockSpec((blk, 1), lambda i: (i, 0)),
        out_shape=jax.ShapeDtypeStruct((_BATCH, 1), jnp.float32),
    )(x, summed, w1t, b1, w2t, b2)


def kernel(x, table, W1, b1, W2, b2):
    xb = x.reshape(_NUM_BLOCKS, _ROWS, _SEQ).swapaxes(1, 2)
    summed = _sc_pool(xb, table)
    return _tc_head(x, summed, W1.T, b1.reshape(1, _EMBED),
                    W2.T, b2.reshape(1, 1))
